# baseline (speedup 1.0000x reference)
import jax
import jax.numpy as jnp
import numpy as np
from jax.experimental import pallas as pl
from jax.experimental.pallas import tpu as pltpu

_NUM_CLUSTERS = 8
_NUM_FEATURES = 512
_N_NODES = 10000
_EPS = 1e-05
_NUM_DEG = 5
_DEN_DEG = 4
_KMEANS_ITERS = 10

_ROWS = 2000
_NB = _N_NODES // _ROWS
_PHASES = _KMEANS_ITERS + 2


def _dot(lhs, rhs, contract):
    return jax.lax.dot_general(
        lhs, rhs, (contract, ((), ())), preferred_element_type=jnp.float32
    )


def _one_hot_labels(x, x_sq, cent):
    dot = _dot(x, cent, (((1,), (1,))))
    c_sq = jnp.sum(cent * cent, axis=1)[None, :]
    d2 = x_sq + c_sq - 2.0 * dot
    min_val = d2[:, 0:1]
    min_idx = jnp.zeros_like(min_val, dtype=jnp.int32)
    for j in range(1, _NUM_CLUSTERS):
        vj = d2[:, j : j + 1]
        take = vj < min_val
        min_val = jnp.where(take, vj, min_val)
        min_idx = jnp.where(take, j, min_idx)
    iota = jax.lax.broadcasted_iota(
        jnp.int32, (x.shape[0], _NUM_CLUSTERS), 1
    )
    return (min_idx == iota).astype(jnp.float32)


def _cna_kernel(
    c0_ref,
    x_ref,
    a_ref,
    b_ref,
    out_ref,
    cent_ref,
    sums_ref,
    sq_ref,
    cnt_ref,
    mean_ref,
    rstd_ref,
    oh_ref,
):
    p = pl.program_id(0)
    blk = pl.program_id(1)
    last_blk = _NB - 1

    @pl.when((p == 0) & (blk == 0))
    def _init_centroids():
        cent_ref[...] = c0_ref[...]

    @pl.when((blk == 0) & (p < _PHASES - 1))
    def _zero_accumulators():
        sums_ref[...] = jnp.zeros_like(sums_ref)
        cnt_ref[...] = jnp.zeros_like(cnt_ref)
        sq_ref[...] = jnp.zeros_like(sq_ref)

    x = x_ref[...]

    @pl.when(p < _PHASES - 1)
    def _assign_and_accumulate():
        x_sq = jnp.sum(x * x, axis=1, keepdims=True)
        oh = _one_hot_labels(x, x_sq, cent_ref[...])
        ones = jnp.ones((_ROWS, 128), dtype=jnp.float32)
        cnt_ref[...] += _dot(oh, ones, (((0,), (0,))))
        sums_ref[...] += _dot(oh, x, (((0,), (0,))))

        @pl.when(p == _KMEANS_ITERS)
        def _stats_extras():
            sq_ref[...] += _dot(oh, x * x, (((0,), (0,))))
            oh_ref[pl.ds(blk * _ROWS, _ROWS), :] = oh

        @pl.when((p < _KMEANS_ITERS) & (blk == last_blk))
        def _update_centroids():
            cnt = cnt_ref[...][:, 0:1]
            new_c = sums_ref[...] / jnp.maximum(cnt, 1.0)
            cent_ref[...] = jnp.where(cnt > 0.0, new_c, cent_ref[...])

        @pl.when((p == _KMEANS_ITERS) & (blk == last_blk))
        def _finalize_stats():
            cnt = jnp.maximum(cnt_ref[...][:, 0:1], 1.0)
            mean = sums_ref[...] / cnt
            sq_mean = sq_ref[...] / cnt
            var = jnp.maximum(sq_mean - mean * mean, 0.0)
            mean_ref[...] = mean
            rstd_ref[...] = jax.lax.rsqrt(var + _EPS)

    @pl.when(p == _PHASES - 1)
    def _apply():
        oh = oh_ref[pl.ds(blk * _ROWS, _ROWS), :]
        mean_n = _dot(oh, mean_ref[...], (((1,), (0,))))
        rstd_n = _dot(oh, rstd_ref[...], (((1,), (0,))))
        z = (x - mean_n) * rstd_n
        a_n = _dot(oh, a_ref[...], (((1,), (0,))))
        b_n = _dot(oh, b_ref[...], (((1,), (0,))))
        num = a_n[:, _NUM_DEG : _NUM_DEG + 1] * z + a_n[:, _NUM_DEG - 1 : _NUM_DEG]
        for k in range(_NUM_DEG - 2, -1, -1):
            num = num * z + a_n[:, k : k + 1]
        den_s = b_n[:, _DEN_DEG - 1 : _DEN_DEG]
        for k in range(_DEN_DEG - 2, -1, -1):
            den_s = den_s * z + b_n[:, k : k + 1]
        den_s = den_s * z
        out_ref[...] = num / (1.0 + jnp.abs(den_s))


def kernel(x, a, b):
    init_idx = np.linspace(0, _N_NODES - 1, _NUM_CLUSTERS).astype(np.int32)
    c0 = x[init_idx]

    grid = (_PHASES, _NB)
    out = pl.pallas_call(
        _cna_kernel,
        grid=grid,
        in_specs=[
            pl.BlockSpec(
                (_NUM_CLUSTERS, _NUM_FEATURES), lambda p, blk: (0, 0)
            ),
            pl.BlockSpec((_ROWS, _NUM_FEATURES), lambda p, blk: (blk, 0)),
            pl.BlockSpec((_NUM_CLUSTERS, _NUM_DEG + 1), lambda p, blk: (0, 0)),
            pl.BlockSpec((_NUM_CLUSTERS, _DEN_DEG), lambda p, blk: (0, 0)),
        ],
        out_specs=pl.BlockSpec(
            (_ROWS, _NUM_FEATURES),
            lambda p, blk: (blk * (p // (_PHASES - 1)), 0),
        ),
        out_shape=jax.ShapeDtypeStruct((_N_NODES, _NUM_FEATURES), jnp.float32),
        scratch_shapes=[
            pltpu.VMEM((_NUM_CLUSTERS, _NUM_FEATURES), jnp.float32),
            pltpu.VMEM((_NUM_CLUSTERS, _NUM_FEATURES), jnp.float32),
            pltpu.VMEM((_NUM_CLUSTERS, _NUM_FEATURES), jnp.float32),
            pltpu.VMEM((_NUM_CLUSTERS, 128), jnp.float32),
            pltpu.VMEM((_NUM_CLUSTERS, _NUM_FEATURES), jnp.float32),
            pltpu.VMEM((_NUM_CLUSTERS, _NUM_FEATURES), jnp.float32),
            pltpu.VMEM((_N_NODES, _NUM_CLUSTERS), jnp.float32),
        ],
        compiler_params=pltpu.CompilerParams(
            dimension_semantics=("arbitrary", "arbitrary"),
        ),
    )(c0, x, a, b)
    return out


# speedup vs baseline: 1.8048x; 1.8048x over previous
import jax
import jax.numpy as jnp
import numpy as np
from jax.experimental import pallas as pl
from jax.experimental.pallas import tpu as pltpu

_NUM_CLUSTERS = 8
_NUM_FEATURES = 512
_N_NODES = 10000
_EPS = 1e-05
_NUM_DEG = 5
_DEN_DEG = 4
_KMEANS_ITERS = 10

_ROWS = 2000
_NB = _N_NODES // _ROWS
_PHASES = _KMEANS_ITERS + 2


def _dot(lhs, rhs, contract):
    return jax.lax.dot_general(
        lhs, rhs, (contract, ((), ())), preferred_element_type=jnp.float32
    )


def _labels(x, x_sq, cent):
    dot = _dot(x, cent, (((1,), (1,))))
    c_sq = jnp.sum(cent * cent, axis=1)[None, :]
    d2 = x_sq + c_sq - 2.0 * dot
    rows = x.shape[0]
    iota = jax.lax.broadcasted_iota(
        jnp.int32, (rows, _NUM_CLUSTERS), 1
    ).astype(jnp.float32)
    min_v = jnp.min(d2, axis=1, keepdims=True)
    labf = jnp.min(
        jnp.where(d2 == min_v, iota, float(_NUM_CLUSTERS)),
        axis=1,
        keepdims=True,
    )
    oh = (iota == labf).astype(jnp.float32)
    return oh, labf


def _cna_kernel(
    c0_ref,
    x_ref,
    a_ref,
    b_ref,
    out_ref,
    cent_ref,
    sums_ref,
    sq_ref,
    cnt_ref,
    mean_ref,
    rstd_ref,
    lab_ref,
):
    p = pl.program_id(0)
    blk = pl.program_id(1)
    last_blk = _NB - 1
    id8 = jnp.eye(_NUM_CLUSTERS, dtype=jnp.float32)

    @pl.when((p == 0) & (blk == 0))
    def _init_centroids():
        cent_ref[...] = c0_ref[...]

    @pl.when((blk == 0) & (p < _PHASES - 1))
    def _zero_accumulators():
        sums_ref[...] = jnp.zeros_like(sums_ref)
        cnt_ref[...] = jnp.zeros_like(cnt_ref)
        sq_ref[...] = jnp.zeros_like(sq_ref)

    x = x_ref[...]

    @pl.when(p < _PHASES - 1)
    def _assign_and_accumulate():
        x_sq = jnp.sum(x * x, axis=1, keepdims=True)
        oh, labf = _labels(x, x_sq, cent_ref[...])
        cnt_ref[...] += jnp.sum(oh, axis=0, keepdims=True)
        sums_ref[...] += _dot(oh, x, (((0,), (0,))))

        @pl.when(p == _KMEANS_ITERS)
        def _stats_extras():
            sq_ref[...] += _dot(oh, x * x, (((0,), (0,))))
            lab_ref[pl.ds(blk * _ROWS, _ROWS), :] = labf

        @pl.when((p < _KMEANS_ITERS) & (blk == last_blk))
        def _update_centroids():
            cnt = _dot(id8, cnt_ref[...], (((1,), (1,))))
            new_c = sums_ref[...] / jnp.maximum(cnt, 1.0)
            cent_ref[...] = jnp.where(cnt > 0.0, new_c, cent_ref[...])

        @pl.when((p == _KMEANS_ITERS) & (blk == last_blk))
        def _finalize_stats():
            cnt = jnp.maximum(
                _dot(id8, cnt_ref[...], (((1,), (1,)))), 1.0
            )
            mean = sums_ref[...] / cnt
            sq_mean = sq_ref[...] / cnt
            var = jnp.maximum(sq_mean - mean * mean, 0.0)
            mean_ref[...] = mean
            rstd_ref[...] = jax.lax.rsqrt(var + _EPS)

    @pl.when(p == _PHASES - 1)
    def _apply():
        labf = lab_ref[pl.ds(blk * _ROWS, _ROWS), :]
        mean = mean_ref[...]
        rstd = rstd_ref[...]
        av = a_ref[...]
        bv = b_ref[...]
        m0 = (labf == 0.0).astype(jnp.float32)
        mean_n = m0 * mean[0:1, :]
        rstd_n = m0 * rstd[0:1, :]
        a_n = m0 * av[0:1, :]
        b_n = m0 * bv[0:1, :]
        for j in range(1, _NUM_CLUSTERS):
            mj = (labf == float(j)).astype(jnp.float32)
            mean_n = mean_n + mj * mean[j : j + 1, :]
            rstd_n = rstd_n + mj * rstd[j : j + 1, :]
            a_n = a_n + mj * av[j : j + 1, :]
            b_n = b_n + mj * bv[j : j + 1, :]
        z = (x - mean_n) * rstd_n
        num = a_n[:, _NUM_DEG : _NUM_DEG + 1] * z + a_n[:, _NUM_DEG - 1 : _NUM_DEG]
        for k in range(_NUM_DEG - 2, -1, -1):
            num = num * z + a_n[:, k : k + 1]
        den_s = b_n[:, _DEN_DEG - 1 : _DEN_DEG]
        for k in range(_DEN_DEG - 2, -1, -1):
            den_s = den_s * z + b_n[:, k : k + 1]
        den_s = den_s * z
        out_ref[...] = num / (1.0 + jnp.abs(den_s))


def kernel(x, a, b):
    init_idx = np.linspace(0, _N_NODES - 1, _NUM_CLUSTERS).astype(np.int32)
    c0 = x[init_idx]

    grid = (_PHASES, _NB)
    out = pl.pallas_call(
        _cna_kernel,
        grid=grid,
        in_specs=[
            pl.BlockSpec(
                (_NUM_CLUSTERS, _NUM_FEATURES), lambda p, blk: (0, 0)
            ),
            pl.BlockSpec((_ROWS, _NUM_FEATURES), lambda p, blk: (blk, 0)),
            pl.BlockSpec((_NUM_CLUSTERS, _NUM_DEG + 1), lambda p, blk: (0, 0)),
            pl.BlockSpec((_NUM_CLUSTERS, _DEN_DEG), lambda p, blk: (0, 0)),
        ],
        out_specs=pl.BlockSpec(
            (_ROWS, _NUM_FEATURES),
            lambda p, blk: (blk * (p // (_PHASES - 1)), 0),
        ),
        out_shape=jax.ShapeDtypeStruct((_N_NODES, _NUM_FEATURES), jnp.float32),
        scratch_shapes=[
            pltpu.VMEM((_NUM_CLUSTERS, _NUM_FEATURES), jnp.float32),
            pltpu.VMEM((_NUM_CLUSTERS, _NUM_FEATURES), jnp.float32),
            pltpu.VMEM((_NUM_CLUSTERS, _NUM_FEATURES), jnp.float32),
            pltpu.VMEM((1, _NUM_CLUSTERS), jnp.float32),
            pltpu.VMEM((_NUM_CLUSTERS, _NUM_FEATURES), jnp.float32),
            pltpu.VMEM((_NUM_CLUSTERS, _NUM_FEATURES), jnp.float32),
            pltpu.VMEM((_N_NODES, 1), jnp.float32),
        ],
        compiler_params=pltpu.CompilerParams(
            dimension_semantics=("arbitrary", "arbitrary"),
        ),
    )(c0, x, a, b)
    return out
